# Initial kernel scaffold; baseline (speedup 1.0000x reference)
import jax
import jax.numpy as jnp
from jax.experimental import pallas as pl


def kernel(batch, table, pos_enc):
    raise NotImplementedError("write your pallas kernel here")


# speedup vs baseline: 1.9874x; 1.9874x over previous
import functools
import math

import jax
import jax.numpy as jnp
from jax import lax
from jax.experimental import pallas as pl
from jax.experimental.pallas import tpu as pltpu
from jax.experimental.pallas import tpu_sc as plsc

D = 128
SEQ = 200
NC, NS = 2, 16
NW = NC * NS
CHUNK = 128
SQRT_D = math.sqrt(D)


def _make_kernel(n_rows):
    rows_per_w = n_rows // NW
    n_chunks = rows_per_w // CHUNK

    @functools.partial(
        pl.kernel,
        out_type=jax.ShapeDtypeStruct((n_rows, D), jnp.float32),
        mesh=plsc.VectorSubcoreMesh(
            core_axis_name="c", subcore_axis_name="s",
            num_cores=NC, num_subcores=NS),
        scratch_types=[
            pltpu.VMEM((CHUNK,), jnp.int32),
            pltpu.VMEM((CHUNK, D), jnp.float32),
            pltpu.VMEM((SEQ, D), jnp.float32),
            pltpu.SemaphoreType.DMA,
        ],
    )
    def emb_kernel(batch_hbm, table_hbm, pos_hbm, out_hbm,
                   idx_v, buf_v, pos_v, sem):
        wid = lax.axis_index("s") * NC + lax.axis_index("c")
        base_row = wid * rows_per_w
        pltpu.sync_copy(pos_hbm, pos_v)

        def chunk_body(c, carry):
            off = base_row + c * CHUNK
            pltpu.sync_copy(batch_hbm.at[pl.ds(off, CHUNK)], idx_v)
            pltpu.async_copy(table_hbm.at[idx_v], buf_v, sem).wait()
            pbase = lax.rem(c * CHUNK, SEQ)

            def group_body(g, carry2):
                iv = idx_v[pl.ds(g * 16, 16)]
                sv = jnp.where(iv == 0, 0.0, SQRT_D).astype(jnp.float32)
                for k in range(16):
                    i = g * 16 + k
                    sck = sv.at[jnp.full((16,), k, jnp.int32)].get(
                        mode="promise_in_bounds")
                    p = pbase + i
                    p = jnp.where(p >= SEQ, p - SEQ, p)
                    for j in range(D // 16):
                        sl = pl.ds(j * 16, 16)
                        buf_v[i, sl] = buf_v[i, sl] * sck + pos_v[p, sl]
                return carry2

            lax.fori_loop(0, CHUNK // 16, group_body, 0)
            pltpu.sync_copy(buf_v, out_hbm.at[pl.ds(off, CHUNK)])
            return carry

        lax.fori_loop(0, n_chunks, chunk_body, 0)

    return emb_kernel


def kernel(batch, table, pos_enc):
    b, s = batch.shape
    idx_flat = jnp.asarray(batch, jnp.int32).reshape(-1)
    out = _make_kernel(b * s)(idx_flat, table, pos_enc)
    return out.reshape(b, s, D)
